# Initial kernel scaffold; baseline (speedup 1.0000x reference)
import jax
import jax.numpy as jnp
from jax.experimental import pallas as pl


def kernel(token_ids, W):
    raise NotImplementedError("write your pallas kernel here")


# speedup vs baseline: 3.6414x; 3.6414x over previous
import functools
import jax
import jax.numpy as jnp
from jax import lax
from jax.experimental import pallas as pl
from jax.experimental.pallas import tpu as pltpu
from jax.experimental.pallas import tpu_sc as plsc

VOCAB = 100000
DIM = 64
PAD = 0

NC = 2
NS = 16
NW = NC * NS

B = 1024 * 20 * 20
BPW = B // NW
UNIT = 128
NU = BPW // UNIT


def _emb_body(idx_hbm, table_hbm, out_hbm, idx_v, rows_v, sem):
    wid = lax.axis_index("s") * NC + lax.axis_index("c")
    base = wid * BPW
    pltpu.sync_copy(idx_hbm.at[pl.ds(base, BPW)], idx_v)

    def unit_body(u, carry):
        uoff = u * UNIT
        pltpu.async_copy(table_hbm.at[idx_v.at[pl.ds(uoff, UNIT)]],
                         rows_v, sem).wait()

        def group_fix(g, c2):
            goff = uoff + g * 16
            iv = idx_v[pl.ds(goff, 16)]
            has_pad = jnp.min(iv, axis=0) == PAD

            @pl.when(has_pad)
            def _():
                m = iv == PAD
                row_idx = g * 16 + lax.iota(jnp.int32, 16)
                zeros = jnp.zeros((16,), jnp.float32)
                for c in range(DIM):
                    col_idx = jnp.full((16,), c, jnp.int32)
                    plsc.store_scatter(rows_v, [row_idx, col_idx], zeros,
                                       mask=m)
            return c2
        lax.fori_loop(0, UNIT // 16, group_fix, 0)

        pltpu.sync_copy(rows_v, out_hbm.at[pl.ds(base + uoff, UNIT)])
        return carry

    lax.fori_loop(0, NU, unit_body, 0)


@functools.partial(jax.jit, static_argnames=())
def _run(idx_flat, W):
    mesh = plsc.VectorSubcoreMesh(core_axis_name="c", subcore_axis_name="s")
    f = pl.kernel(
        _emb_body,
        out_type=jax.ShapeDtypeStruct((B, DIM), jnp.float32),
        mesh=mesh,
        scratch_types=[
            pltpu.VMEM((BPW,), jnp.int32),
            pltpu.VMEM((UNIT, DIM), jnp.float32),
            pltpu.SemaphoreType.DMA,
        ],
        compiler_params=pltpu.CompilerParams(
            needs_layout_passes=False, use_tc_tiling_on_sc=False),
    )
    return f(idx_flat, W)


def kernel(token_ids, W):
    bsz, seq, inner = token_ids.shape
    idx_flat = token_ids.reshape(-1).astype(jnp.int32)
    out = _run(idx_flat, W)
    return out.reshape(bsz, seq, inner, DIM)


# speedup vs baseline: 4.4155x; 1.2126x over previous
import functools
import jax
import jax.numpy as jnp
from jax import lax
from jax.experimental import pallas as pl
from jax.experimental.pallas import tpu as pltpu
from jax.experimental.pallas import tpu_sc as plsc

VOCAB = 100000
DIM = 64
PAD = 0

NC = 2
NS = 16
NW = NC * NS

B = 1024 * 20 * 20
BPW = B // NW
UNIT = 128
NU = BPW // UNIT
NB = 10
G = 5


def _emb_body(idx_hbm, table_hbm, out_hbm, idx_v, rows_v, gsem, osem):
    wid = lax.axis_index("s") * NC + lax.axis_index("c")
    base = wid * BPW
    pltpu.sync_copy(idx_hbm.at[pl.ds(base, BPW)], idx_v)

    def gather_copy(u, b):
        return pltpu.make_async_copy(
            table_hbm.at[idx_v.at[pl.ds(u * UNIT, UNIT)]],
            rows_v.at[b], gsem.at[b])

    def out_copy(u, b):
        return pltpu.make_async_copy(
            rows_v.at[b], out_hbm.at[pl.ds(base + u * UNIT, UNIT)],
            osem.at[b])

    def fixup(u, b):
        def group_fix(g, c2):
            goff = u * UNIT + g * 16
            iv = idx_v[pl.ds(goff, 16)]
            has_pad = jnp.min(iv, axis=0) == PAD

            @pl.when(has_pad)
            def _():
                m = iv == PAD
                row_idx = g * 16 + lax.iota(jnp.int32, 16)
                zeros = jnp.zeros((16,), jnp.float32)
                for c in range(DIM):
                    col_idx = jnp.full((16,), c, jnp.int32)
                    plsc.store_scatter(rows_v.at[b], [row_idx, col_idx],
                                       zeros, mask=m)
            return c2
        lax.fori_loop(0, UNIT // 16, group_fix, 0)

    for u in range(G):
        gather_copy(u, u).start()

    def round_body(t, carry):
        for b in range(NB):
            u = t * NB + b
            up = u + G
            bp = (b + G) % NB

            @pl.when(jnp.logical_and(up < NU, up >= NB))
            def _():
                out_copy(up - NB, bp).wait()

            @pl.when(up < NU)
            def _():
                gather_copy(up, bp).start()

            gather_copy(u, b).wait()
            fixup(u, b)
            out_copy(u, b).start()
        return carry

    lax.fori_loop(0, NU // NB, round_body, 0)

    for b in range(NB):
        out_copy((NU // NB - 1) * NB + b, b).wait()


@functools.partial(jax.jit, static_argnames=())
def _run(idx_flat, W):
    mesh = plsc.VectorSubcoreMesh(core_axis_name="c", subcore_axis_name="s")
    f = pl.kernel(
        _emb_body,
        out_type=jax.ShapeDtypeStruct((B, DIM), jnp.float32),
        mesh=mesh,
        scratch_types=[
            pltpu.VMEM((BPW,), jnp.int32),
            pltpu.VMEM((NB, UNIT, DIM), jnp.float32),
            pltpu.SemaphoreType.DMA((NB,)),
            pltpu.SemaphoreType.DMA((NB,)),
        ],
        compiler_params=pltpu.CompilerParams(
            needs_layout_passes=False, use_tc_tiling_on_sc=False),
    )
    return f(idx_flat, W)


def kernel(token_ids, W):
    bsz, seq, inner = token_ids.shape
    idx_flat = token_ids.reshape(-1).astype(jnp.int32)
    out = _run(idx_flat, W)
    return out.reshape(bsz, seq, inner, DIM)
